# Initial kernel scaffold; baseline (speedup 1.0000x reference)
import functools

import jax
import jax.numpy as jnp
from jax import lax
from jax.experimental import pallas as pl
from jax.experimental.pallas import tpu as pltpu
from jax.experimental.pallas import tpu_sc as plsc

_D = 32
_ROWS = 128
_NBUF = 4
_NC = 2
_NS = 16


@functools.lru_cache(maxsize=None)
def _make_gather(n_chunks: int):
    steps = n_chunks // (_NC * _NS)
    groups = steps // _NBUF
    assert steps % _NBUF == 0 and n_chunks % (_NC * _NS) == 0
    mesh = plsc.VectorSubcoreMesh(core_axis_name="c", subcore_axis_name="s")

    @functools.partial(
        pl.kernel,
        mesh=mesh,
        out_type=jax.ShapeDtypeStruct((n_chunks, _ROWS, _D), jnp.float32),
        scratch_types=(
            [
                pltpu.VMEM((steps, _ROWS), jnp.int32),
                pltpu.VMEM((_NBUF, _ROWS, _D), jnp.float32),
            ]
            + [pltpu.SemaphoreType.DMA] * (2 * _NBUF)
        ),
    )
    def gather(table_hbm, idx_hbm, out_hbm, idx_v, bufs, *sems):
        gsem = sems[:_NBUF]
        osem = sems[_NBUF:]
        wid = lax.axis_index("s") * _NC + lax.axis_index("c")
        base = wid * steps
        pltpu.sync_copy(idx_hbm.at[pl.ds(base, steps)], idx_v)
        for b in range(_NBUF):
            pltpu.async_copy(table_hbm.at[idx_v.at[b]], bufs.at[b], gsem[b])

        def group(g, carry):
            for b in range(_NBUF):
                s = g * _NBUF + b
                pltpu.make_async_copy(
                    table_hbm.at[idx_v.at[s]], bufs.at[b], gsem[b]
                ).wait()
                pltpu.async_copy(bufs.at[b], out_hbm.at[base + s], osem[b])
                pltpu.make_async_copy(
                    bufs.at[b], out_hbm.at[base + s], osem[b]
                ).wait()
                pltpu.async_copy(
                    table_hbm.at[idx_v.at[s + _NBUF]], bufs.at[b], gsem[b]
                )
            return carry

        lax.fori_loop(0, groups - 1, group, 0)
        for b in range(_NBUF):
            s = (groups - 1) * _NBUF + b
            pltpu.make_async_copy(
                table_hbm.at[idx_v.at[s]], bufs.at[b], gsem[b]
            ).wait()
            pltpu.sync_copy(bufs.at[b], out_hbm.at[base + s])

    return gather


def kernel(word_inputs, word_seq_lengths, word_embedding):
    del word_seq_lengths
    b, s = word_inputs.shape
    _, d = word_embedding.shape
    n_chunks = (b * s) // _ROWS
    idx = word_inputs.astype(jnp.int32).reshape(n_chunks, _ROWS)
    out = _make_gather(n_chunks)(word_embedding, idx)
    return out.reshape(b, s, d)


# speedup vs baseline: 1.4926x; 1.4926x over previous
import functools

import jax
import jax.numpy as jnp
from jax import lax
from jax.experimental import pallas as pl
from jax.experimental.pallas import tpu as pltpu
from jax.experimental.pallas import tpu_sc as plsc

_D = 32
_ROWS = 128
_NBUF = 4
_NC = 2
_NS = 16


@functools.lru_cache(maxsize=None)
def _make_gather(n_chunks: int):
    steps = n_chunks // (_NC * _NS)
    groups = steps // _NBUF
    assert steps % _NBUF == 0 and n_chunks % (_NC * _NS) == 0
    mesh = plsc.VectorSubcoreMesh(core_axis_name="c", subcore_axis_name="s")

    @functools.partial(
        pl.kernel,
        mesh=mesh,
        out_type=jax.ShapeDtypeStruct((n_chunks, _ROWS, _D), jnp.float32),
        scratch_types=(
            [
                pltpu.VMEM((steps, _ROWS), jnp.int32),
                pltpu.VMEM((_NBUF, _ROWS, _D), jnp.float32),
            ]
            + [pltpu.SemaphoreType.DMA] * (2 * _NBUF)
        ),
        compiler_params=pltpu.CompilerParams(use_tc_tiling_on_sc=False),
    )
    def gather(table_hbm, idx_hbm, out_hbm, idx_v, bufs, *sems):
        gsem = sems[:_NBUF]
        osem = sems[_NBUF:]
        wid = lax.axis_index("s") * _NC + lax.axis_index("c")
        base = wid * steps
        pltpu.sync_copy(idx_hbm.at[pl.ds(base, steps)], idx_v)
        for b in range(_NBUF):
            pltpu.async_copy(table_hbm.at[idx_v.at[b]], bufs.at[b], gsem[b])

        def group(g, carry):
            for b in range(_NBUF):
                s = g * _NBUF + b
                pltpu.make_async_copy(
                    table_hbm.at[idx_v.at[s]], bufs.at[b], gsem[b]
                ).wait()
                pltpu.async_copy(bufs.at[b], out_hbm.at[base + s], osem[b])
                pltpu.make_async_copy(
                    bufs.at[b], out_hbm.at[base + s], osem[b]
                ).wait()
                pltpu.async_copy(
                    table_hbm.at[idx_v.at[s + _NBUF]], bufs.at[b], gsem[b]
                )
            return carry

        lax.fori_loop(0, groups - 1, group, 0)
        for b in range(_NBUF):
            s = (groups - 1) * _NBUF + b
            pltpu.make_async_copy(
                table_hbm.at[idx_v.at[s]], bufs.at[b], gsem[b]
            ).wait()
            pltpu.sync_copy(bufs.at[b], out_hbm.at[base + s])

    return gather


def kernel(word_inputs, word_seq_lengths, word_embedding):
    del word_seq_lengths
    b, s = word_inputs.shape
    _, d = word_embedding.shape
    n_chunks = (b * s) // _ROWS
    idx = word_inputs.astype(jnp.int32).reshape(n_chunks, _ROWS)
    out = _make_gather(n_chunks)(word_embedding, idx)
    return out.reshape(b, s, d)
